# baseline (speedup 1.0000x reference)
import functools

import jax
import jax.numpy as jnp
from jax import lax
from jax.experimental import pallas as pl
from jax.experimental.pallas import tpu as pltpu
from jax.experimental.pallas import tpu_sc as plsc

TOPK = 5
KPAD = 8
CW = 48


def _topk_tc_kernel(e_full_ref, a_full_ref, m_ref, val_ref, idx_ref, af_ref,
                    cand_v_ref, cand_i_ref, *, blk, n, topk):
    nt = n // blk
    dn = (((1,), (1,)), ((), ()))
    col_loc = lax.broadcasted_iota(jnp.int32, (blk, blk), 0)
    row_loc = lax.broadcasted_iota(jnp.int32, (blk, blk), 1)

    d = a_full_ref.shape[1]
    m = m_ref[m_ref.shape[0] - 1]
    dcol = lax.broadcasted_iota(jnp.int32, (n, d), 1)
    af_ref[...] = jnp.where(dcol < m, a_full_ref[...], 0.0)

    for r in range(nt):
        e_blk = e_full_ref[r * blk:(r + 1) * blk, :]
        cand_v_ref[...] = jnp.full((CW, blk), -jnp.inf, jnp.float32)
        cand_i_ref[...] = jnp.zeros((CW, blk), jnp.int32)
        if r > 0:
            s0 = nt * topk
            cand_v_ref[s0:s0 + topk, :] = jnp.zeros((topk, blk), jnp.float32)
            cand_i_ref[s0:s0 + topk, :] = lax.broadcasted_iota(
                jnp.int32, (topk, blk), 0)

        for c in range(r, nt):
            S = lax.dot_general(e_full_ref[c * blk:(c + 1) * blk, :], e_blk,
                                dn, precision=lax.Precision.HIGHEST,
                                preferred_element_type=jnp.float32)
            if c == r:
                S = jnp.where(col_loc > row_loc, S, 0.0)
            if r > 0:
                b = lax.bitcast_convert_type(jnp.maximum(S, 0.0), jnp.int32)
                key = (b & jnp.int32(-512)) | (jnp.int32(blk - 1) - col_loc)
            else:
                b = lax.bitcast_convert_type(S, jnp.int32)
                key = b ^ (lax.shift_right_arithmetic(b, 31)
                           & jnp.int32(0x7FFFFFFF))
                key = (key & jnp.int32(-512)) | (jnp.int32(blk - 1) - col_loc)
            for k in range(topk):
                mk = jnp.max(key, axis=0, keepdims=True)
                s = c * topk + k
                mkc = mk & jnp.int32(-512)
                if r > 0:
                    vbits = mkc
                else:
                    vbits = mkc ^ (lax.shift_right_arithmetic(mkc, 31)
                                   & jnp.int32(0x7FFFFFFF))
                cand_v_ref[s:s + 1, :] = lax.bitcast_convert_type(
                    vbits, jnp.float32)
                cand_i_ref[s:s + 1, :] = (c * blk + (blk - 1)) - (mk & jnp.int32(511))
                if k + 1 < topk:
                    key = jnp.where(key == mk, jnp.int32(-2147483648), key)

        CV = cand_v_ref[...]
        CI = cand_i_ref[...]
        for k in range(topk):
            mm = jnp.max(CV, axis=0, keepdims=True)
            jsel = jnp.min(jnp.where(CV == mm, CI, n), axis=0, keepdims=True)
            val_ref[k:k + 1, r * blk:(r + 1) * blk] = mm
            idx_ref[k:k + 1, r * blk:(r + 1) * blk] = jsel
            if k + 1 < topk:
                CV = jnp.where((CV == mm) & (CI == jsel), -jnp.inf, CV)


def _run_tc_topk(embeddings, adapted, m_list, n, d, blk):
    return pl.pallas_call(
        functools.partial(_topk_tc_kernel, blk=blk, n=n, topk=TOPK),
        in_specs=[
            pl.BlockSpec(),
            pl.BlockSpec(),
            pl.BlockSpec(memory_space=pltpu.SMEM),
        ],
        out_shape=(
            jax.ShapeDtypeStruct((TOPK, n), jnp.float32),
            jax.ShapeDtypeStruct((TOPK, n), jnp.int32),
            jax.ShapeDtypeStruct((n, d), jnp.float32),
        ),
        scratch_shapes=[
            pltpu.VMEM((CW, blk), jnp.float32),
            pltpu.VMEM((CW, blk), jnp.int32),
        ],
    )(embeddings, adapted, m_list)


def _pairs_sc_kernel(a2_hbm, idxq_hbm, valq_hbm, s_out, c_out,
                     own_v, gath_v, idxf_v, valf_v, s_stage, c_stage, sem,
                     *, d, n, topk, rows_per_w):
    wid = lax.axis_index("s") * 2 + lax.axis_index("c")
    base_row = wid * rows_per_w
    ppw = rows_per_w * topk

    stage = [pltpu.async_copy(a2_hbm.at[pl.ds(base_row, rows_per_w)],
                              own_v, sem)]
    for k in range(topk):
        stage.append(pltpu.async_copy(
            idxq_hbm.at[pl.ds(k * n + base_row, rows_per_w)],
            idxf_v.at[pl.ds(k * rows_per_w, rows_per_w)], sem))
        stage.append(pltpu.async_copy(
            valq_hbm.at[pl.ds(k * n + base_row, rows_per_w)],
            valf_v.at[pl.ds(k * rows_per_w, rows_per_w)], sem))
    for h in stage:
        h.wait()

    nchunks = ppw // 128
    handles = []
    for c in range(nchunks):
        handles.append(pltpu.async_copy(
            a2_hbm.at[idxf_v.at[pl.ds(c * 128, 128)]],
            gath_v.at[pl.ds(c * 128, 128)], sem))
    for h in handles:
        h.wait()

    lane = lax.broadcasted_iota(jnp.int32, (16,), 0)

    def one_group(kbase, s_acc, c_acc):
        jv = idxf_v[pl.ds(kbase, 16)]
        vv = valf_v[pl.ds(kbase, 16)]
        off = kbase - ((kbase // rows_per_w) * rows_per_w)
        i_loc = off + lane
        iv = base_row + i_loc
        pairidx = kbase + lane
        acc0 = jnp.zeros((16,), jnp.float32)
        acc1 = jnp.zeros((16,), jnp.float32)
        for dd in range(0, d, 2):
            d0 = jnp.full((16,), dd, jnp.int32)
            d1 = jnp.full((16,), dd + 1, jnp.int32)
            acc0 = acc0 + (plsc.load_gather(own_v, [i_loc, d0]) *
                           plsc.load_gather(gath_v, [pairidx, d0]))
            acc1 = acc1 + (plsc.load_gather(own_v, [i_loc, d1]) *
                           plsc.load_gather(gath_v, [pairidx, d1]))
        red = jnp.where(jv > iv, acc0 + acc1, 0.0)
        s_acc = s_acc + jnp.abs(vv - red)
        c_acc = c_acc + jnp.where(vv != 0.0, 1.0, 0.0)
        return s_acc, c_acc

    def body(g, carry):
        s_acc, c_acc = carry
        for u in range(4):
            s_acc, c_acc = one_group(g * 64 + u * 16, s_acc, c_acc)
        return s_acc, c_acc

    zero = jnp.zeros((16,), jnp.float32)
    s_acc, c_acc = lax.fori_loop(0, ppw // 64, body, (zero, zero))

    s_stage[...] = s_acc
    c_stage[...] = c_acc
    pltpu.sync_copy(s_stage, s_out.at[wid])
    pltpu.sync_copy(c_stage, c_out.at[wid])


def kernel(embeddings, adapted_embeddings, m_list):
    n, d = embeddings.shape
    blk = 512
    vals_t, idxs_t, a_masked = _run_tc_topk(
        embeddings, adapted_embeddings, m_list.astype(jnp.int32), n, d, blk)

    nw = 32
    rows_per_w = n // nw
    ppw = rows_per_w * TOPK
    idxq = idxs_t.reshape(TOPK * n)
    valq = vals_t.reshape(TOPK * n)

    mesh = plsc.VectorSubcoreMesh(core_axis_name="c", subcore_axis_name="s")
    sc = pl.kernel(
        functools.partial(_pairs_sc_kernel, d=d, n=n, topk=TOPK,
                          rows_per_w=rows_per_w),
        mesh=mesh,
        compiler_params=pltpu.CompilerParams(needs_layout_passes=False,
                                             use_tc_tiling_on_sc=False),
        out_type=(
            jax.ShapeDtypeStruct((nw, 16), jnp.float32),
            jax.ShapeDtypeStruct((nw, 16), jnp.float32),
        ),
        scratch_types=[
            pltpu.VMEM((rows_per_w, d), jnp.float32),
            pltpu.VMEM((ppw, d), jnp.float32),
            pltpu.VMEM((ppw,), jnp.int32),
            pltpu.VMEM((ppw,), jnp.float32),
            pltpu.VMEM((16,), jnp.float32),
            pltpu.VMEM((16,), jnp.float32),
            pltpu.SemaphoreType.DMA,
        ],
    )
    s_part, c_part = sc(a_masked, idxq, valq)

    loss = jnp.sum(s_part) / jnp.float32(n * n)
    return loss / jnp.sum(c_part)


# speedup vs baseline: 1.0279x; 1.0279x over previous
import functools

import jax
import jax.numpy as jnp
from jax import lax
from jax.experimental import pallas as pl
from jax.experimental.pallas import tpu as pltpu
from jax.experimental.pallas import tpu_sc as plsc

TOPK = 5
KPAD = 8
CW = 48


def _topk_tc_kernel(e_full_ref, a_full_ref, m_ref, val_ref, idx_ref, af_ref,
                    cand_v_ref, cand_i_ref, *, blk, n, topk):
    nt = n // blk
    dn = (((1,), (1,)), ((), ()))
    col_loc = lax.broadcasted_iota(jnp.int32, (blk, blk), 0)
    row_loc = lax.broadcasted_iota(jnp.int32, (blk, blk), 1)

    d = a_full_ref.shape[1]
    m = m_ref[m_ref.shape[0] - 1]
    dcol = lax.broadcasted_iota(jnp.int32, (n, d), 1)
    af_ref[...] = jnp.where(dcol < m, a_full_ref[...], 0.0)

    for r in range(nt):
        e_blk = e_full_ref[r * blk:(r + 1) * blk, :]
        cand_v_ref[...] = jnp.full((CW, blk), -jnp.inf, jnp.float32)
        cand_i_ref[...] = jnp.zeros((CW, blk), jnp.int32)
        if r > 0:
            s0 = nt * topk
            cand_v_ref[s0:s0 + topk, :] = jnp.zeros((topk, blk), jnp.float32)
            cand_i_ref[s0:s0 + topk, :] = lax.broadcasted_iota(
                jnp.int32, (topk, blk), 0)

        for c in range(r, nt):
            S = lax.dot_general(e_full_ref[c * blk:(c + 1) * blk, :], e_blk,
                                dn, precision=lax.Precision.HIGHEST,
                                preferred_element_type=jnp.float32)
            if c == r:
                S = jnp.where(col_loc > row_loc, S, 0.0)
            if r > 0:
                b = lax.bitcast_convert_type(jnp.maximum(S, 0.0), jnp.int32)
                key = (b & jnp.int32(-512)) | (jnp.int32(blk - 1) - col_loc)
            else:
                b = lax.bitcast_convert_type(S, jnp.int32)
                key = b ^ (lax.shift_right_arithmetic(b, 31)
                           & jnp.int32(0x7FFFFFFF))
                key = (key & jnp.int32(-512)) | (jnp.int32(blk - 1) - col_loc)
            for k in range(topk):
                mk = jnp.max(key, axis=0, keepdims=True)
                s = c * topk + k
                mkc = mk & jnp.int32(-512)
                if r > 0:
                    vbits = mkc
                else:
                    vbits = mkc ^ (lax.shift_right_arithmetic(mkc, 31)
                                   & jnp.int32(0x7FFFFFFF))
                cand_v_ref[s:s + 1, :] = lax.bitcast_convert_type(
                    vbits, jnp.float32)
                cand_i_ref[s:s + 1, :] = (c * blk + (blk - 1)) - (mk & jnp.int32(511))
                if k + 1 < topk:
                    key = jnp.where(key == mk, jnp.int32(-2147483648), key)

        CV = cand_v_ref[...]
        CI = cand_i_ref[...]
        for k in range(topk):
            mm = jnp.max(CV, axis=0, keepdims=True)
            jsel = jnp.min(jnp.where(CV == mm, CI, n), axis=0, keepdims=True)
            val_ref[k:k + 1, r * blk:(r + 1) * blk] = mm
            idx_ref[k:k + 1, r * blk:(r + 1) * blk] = jsel
            if k + 1 < topk:
                CV = jnp.where((CV == mm) & (CI == jsel), -jnp.inf, CV)


def _run_tc_topk(embeddings, adapted, m_list, n, d, blk):
    return pl.pallas_call(
        functools.partial(_topk_tc_kernel, blk=blk, n=n, topk=TOPK),
        in_specs=[
            pl.BlockSpec(),
            pl.BlockSpec(),
            pl.BlockSpec(memory_space=pltpu.SMEM),
        ],
        out_shape=(
            jax.ShapeDtypeStruct((TOPK, n), jnp.float32),
            jax.ShapeDtypeStruct((TOPK, n), jnp.int32),
            jax.ShapeDtypeStruct((n, d), jnp.float32),
        ),
        scratch_shapes=[
            pltpu.VMEM((CW, blk), jnp.float32),
            pltpu.VMEM((CW, blk), jnp.int32),
        ],
    )(embeddings, adapted, m_list)


def _pairs_sc_kernel(a2_hbm, idxq_hbm, valq_hbm, s_out, c_out,
                     own_v, gath_v, idxf_v, valf_v, s_stage, c_stage, sem,
                     *, d, n, topk, rows_per_w):
    wid = lax.axis_index("s") * 2 + lax.axis_index("c")
    base_row = wid * rows_per_w
    ppw = rows_per_w * topk

    stage = [pltpu.async_copy(a2_hbm.at[pl.ds(base_row, rows_per_w)],
                              own_v, sem)]
    for k in range(topk):
        stage.append(pltpu.async_copy(
            idxq_hbm.at[pl.ds(k * n + base_row, rows_per_w)],
            idxf_v.at[pl.ds(k * rows_per_w, rows_per_w)], sem))
        stage.append(pltpu.async_copy(
            valq_hbm.at[pl.ds(k * n + base_row, rows_per_w)],
            valf_v.at[pl.ds(k * rows_per_w, rows_per_w)], sem))
    for h in stage:
        h.wait()

    nchunks = ppw // 128
    handles = []
    for c in range(nchunks):
        handles.append(pltpu.async_copy(
            a2_hbm.at[idxf_v.at[pl.ds(c * 128, 128)]],
            gath_v.at[pl.ds(c * 128, 128)], sem))
    for h in handles:
        h.wait()

    lane = lax.broadcasted_iota(jnp.int32, (16,), 0)

    def one_group(kbase, s_acc, c_acc):
        jv = idxf_v[pl.ds(kbase, 16)]
        vv = valf_v[pl.ds(kbase, 16)]
        off = kbase - ((kbase // rows_per_w) * rows_per_w)
        i_loc = off + lane
        iv = base_row + i_loc
        pairidx = kbase + lane
        acc0 = jnp.zeros((16,), jnp.float32)
        acc1 = jnp.zeros((16,), jnp.float32)
        for dd in range(0, d, 2):
            d0 = jnp.full((16,), dd, jnp.int32)
            d1 = jnp.full((16,), dd + 1, jnp.int32)
            acc0 = acc0 + (plsc.load_gather(own_v, [i_loc, d0]) *
                           plsc.load_gather(gath_v, [pairidx, d0]))
            acc1 = acc1 + (plsc.load_gather(own_v, [i_loc, d1]) *
                           plsc.load_gather(gath_v, [pairidx, d1]))
        red = jnp.where(jv > iv, acc0 + acc1, 0.0)
        s_acc = s_acc + jnp.abs(vv - red)
        c_acc = c_acc + jnp.where(vv != 0.0, 1.0, 0.0)
        return s_acc, c_acc

    def body(g, carry):
        s_acc, c_acc = carry
        s_acc, c_acc = one_group(g * 32, s_acc, c_acc)
        s_acc, c_acc = one_group(g * 32 + 16, s_acc, c_acc)
        return s_acc, c_acc

    zero = jnp.zeros((16,), jnp.float32)
    s_acc, c_acc = lax.fori_loop(0, ppw // 32, body, (zero, zero))

    s_stage[...] = s_acc
    c_stage[...] = c_acc
    pltpu.sync_copy(s_stage, s_out.at[wid])
    pltpu.sync_copy(c_stage, c_out.at[wid])


def kernel(embeddings, adapted_embeddings, m_list):
    n, d = embeddings.shape
    blk = 512
    vals_t, idxs_t, a_masked = _run_tc_topk(
        embeddings, adapted_embeddings, m_list.astype(jnp.int32), n, d, blk)

    nw = 32
    rows_per_w = n // nw
    ppw = rows_per_w * TOPK
    idxq = idxs_t.reshape(TOPK * n)
    valq = vals_t.reshape(TOPK * n)

    mesh = plsc.VectorSubcoreMesh(core_axis_name="c", subcore_axis_name="s")
    sc = pl.kernel(
        functools.partial(_pairs_sc_kernel, d=d, n=n, topk=TOPK,
                          rows_per_w=rows_per_w),
        mesh=mesh,
        compiler_params=pltpu.CompilerParams(needs_layout_passes=False,
                                             use_tc_tiling_on_sc=False),
        out_type=(
            jax.ShapeDtypeStruct((nw, 16), jnp.float32),
            jax.ShapeDtypeStruct((nw, 16), jnp.float32),
        ),
        scratch_types=[
            pltpu.VMEM((rows_per_w, d), jnp.float32),
            pltpu.VMEM((ppw, d), jnp.float32),
            pltpu.VMEM((ppw,), jnp.int32),
            pltpu.VMEM((ppw,), jnp.float32),
            pltpu.VMEM((16,), jnp.float32),
            pltpu.VMEM((16,), jnp.float32),
            pltpu.SemaphoreType.DMA,
        ],
    )
    s_part, c_part = sc(a_masked, idxq, valq)

    loss = jnp.sum(s_part) / jnp.float32(n * n)
    return loss / jnp.sum(c_part)


# speedup vs baseline: 1.0285x; 1.0005x over previous
import functools

import jax
import jax.numpy as jnp
from jax import lax
from jax.experimental import pallas as pl
from jax.experimental.pallas import tpu as pltpu
from jax.experimental.pallas import tpu_sc as plsc

TOPK = 5
CW = 48


def _topk_tc_kernel(e_full_ref, a_full_ref, m_ref, val_ref, idx_ref, af_ref,
                    cand_v_ref, cand_i_ref, *, blk, n, topk):
    nt = n // blk
    dn = (((1,), (1,)), ((), ()))
    col_loc = lax.broadcasted_iota(jnp.int32, (blk, blk), 0)
    row_loc = lax.broadcasted_iota(jnp.int32, (blk, blk), 1)

    d = a_full_ref.shape[1]
    m = m_ref[m_ref.shape[0] - 1]
    dcol = lax.broadcasted_iota(jnp.int32, (n, d), 1)
    af_ref[...] = jnp.where(dcol < m, a_full_ref[...], 0.0)

    for r in range(nt):
        e_blk = e_full_ref[r * blk:(r + 1) * blk, :]
        cand_v_ref[...] = jnp.full((CW, blk), -jnp.inf, jnp.float32)
        cand_i_ref[...] = jnp.zeros((CW, blk), jnp.int32)
        if r > 0:
            s0 = nt * topk
            cand_v_ref[s0:s0 + topk, :] = jnp.zeros((topk, blk), jnp.float32)
            cand_i_ref[s0:s0 + topk, :] = lax.broadcasted_iota(
                jnp.int32, (topk, blk), 0)

        for c in range(r, nt):
            S = lax.dot_general(e_full_ref[c * blk:(c + 1) * blk, :], e_blk,
                                dn, precision=lax.Precision.HIGHEST,
                                preferred_element_type=jnp.float32)
            if c == r:
                S = jnp.where(col_loc > row_loc, S, 0.0)
            if r > 0:
                b = lax.bitcast_convert_type(jnp.maximum(S, 0.0), jnp.int32)
                key = (b & jnp.int32(-512)) | (jnp.int32(blk - 1) - col_loc)
            else:
                b = lax.bitcast_convert_type(S, jnp.int32)
                key = b ^ (lax.shift_right_arithmetic(b, 31)
                           & jnp.int32(0x7FFFFFFF))
                key = (key & jnp.int32(-512)) | (jnp.int32(blk - 1) - col_loc)
            for k in range(topk):
                mk = jnp.max(key, axis=0, keepdims=True)
                s = c * topk + k
                mkc = mk & jnp.int32(-512)
                if r > 0:
                    vbits = mkc
                else:
                    vbits = mkc ^ (lax.shift_right_arithmetic(mkc, 31)
                                   & jnp.int32(0x7FFFFFFF))
                cand_v_ref[s:s + 1, :] = lax.bitcast_convert_type(
                    vbits, jnp.float32)
                cand_i_ref[s:s + 1, :] = (c * blk + (blk - 1)) - (mk & jnp.int32(511))
                if k + 1 < topk:
                    key = jnp.where(key == mk, jnp.int32(-2147483648), key)

        CV = cand_v_ref[...]
        CI = cand_i_ref[...]
        for k in range(topk):
            mm = jnp.max(CV, axis=0, keepdims=True)
            jsel = jnp.min(jnp.where(CV == mm, CI, n), axis=0, keepdims=True)
            val_ref[k:k + 1, r * blk:(r + 1) * blk] = mm
            idx_ref[k:k + 1, r * blk:(r + 1) * blk] = jsel
            if k + 1 < topk:
                CV = jnp.where((CV == mm) & (CI == jsel), -jnp.inf, CV)


def _run_tc_topk(embeddings, adapted, m_list, n, d, blk):
    return pl.pallas_call(
        functools.partial(_topk_tc_kernel, blk=blk, n=n, topk=TOPK),
        in_specs=[
            pl.BlockSpec(),
            pl.BlockSpec(),
            pl.BlockSpec(memory_space=pltpu.SMEM),
        ],
        out_shape=(
            jax.ShapeDtypeStruct((TOPK, n), jnp.float32),
            jax.ShapeDtypeStruct((TOPK, n), jnp.int32),
            jax.ShapeDtypeStruct((n, d), jnp.float32),
        ),
        scratch_shapes=[
            pltpu.VMEM((CW, blk), jnp.float32),
            pltpu.VMEM((CW, blk), jnp.int32),
        ],
    )(embeddings, adapted, m_list)


def _pairs_sc_kernel(a2_hbm, idxq_hbm, valq_hbm, s_out, c_out,
                     own_v, gath_v, idxf_v, valf_v, s_stage, c_stage, sem,
                     *, d, n, topk, rows_per_w):
    wid = lax.axis_index("s") * 2 + lax.axis_index("c")
    base_row = wid * rows_per_w
    ppw = rows_per_w * topk

    stage = [pltpu.async_copy(a2_hbm.at[pl.ds(base_row, rows_per_w)],
                              own_v, sem)]
    for k in range(topk):
        stage.append(pltpu.async_copy(
            idxq_hbm.at[pl.ds(k * n + base_row, rows_per_w)],
            idxf_v.at[pl.ds(k * rows_per_w, rows_per_w)], sem))
        stage.append(pltpu.async_copy(
            valq_hbm.at[pl.ds(k * n + base_row, rows_per_w)],
            valf_v.at[pl.ds(k * rows_per_w, rows_per_w)], sem))
    for h in stage:
        h.wait()

    nchunks = ppw // 128
    handles = []
    for c in range(nchunks):
        handles.append(pltpu.async_copy(
            a2_hbm.at[idxf_v.at[pl.ds(c * 128, 128)]],
            gath_v.at[pl.ds(c * 128, 128)], sem))
    for h in handles:
        h.wait()

    lane = lax.broadcasted_iota(jnp.int32, (16,), 0)

    def one_group(kbase, s_acc, c_acc):
        jv = idxf_v[pl.ds(kbase, 16)]
        vv = valf_v[pl.ds(kbase, 16)]
        off = kbase - ((kbase // rows_per_w) * rows_per_w)
        i_loc = off + lane
        iv = base_row + i_loc
        pairidx = kbase + lane
        acc0 = jnp.zeros((16,), jnp.float32)
        acc1 = jnp.zeros((16,), jnp.float32)
        for dd in range(0, d, 2):
            d0 = jnp.full((16,), dd, jnp.int32)
            d1 = jnp.full((16,), dd + 1, jnp.int32)
            acc0 = acc0 + (plsc.load_gather(own_v, [i_loc, d0]) *
                           plsc.load_gather(gath_v, [pairidx, d0]))
            acc1 = acc1 + (plsc.load_gather(own_v, [i_loc, d1]) *
                           plsc.load_gather(gath_v, [pairidx, d1]))
        red = jnp.where(jv > iv, acc0 + acc1, 0.0)
        s_acc = s_acc + jnp.abs(vv - red)
        c_acc = c_acc + jnp.where(vv != 0.0, 1.0, 0.0)
        return s_acc, c_acc

    def body(g, carry):
        s_acc, c_acc = carry
        s_acc, c_acc = one_group(g * 32, s_acc, c_acc)
        s_acc, c_acc = one_group(g * 32 + 16, s_acc, c_acc)
        return s_acc, c_acc

    zero = jnp.zeros((16,), jnp.float32)
    s_acc, c_acc = lax.fori_loop(0, ppw // 32, body, (zero, zero))

    s_stage[...] = s_acc
    c_stage[...] = c_acc
    pltpu.sync_copy(s_stage, s_out.at[wid])
    pltpu.sync_copy(c_stage, c_out.at[wid])


def kernel(embeddings, adapted_embeddings, m_list):
    n, d = embeddings.shape
    blk = 512
    vals_t, idxs_t, a_masked = _run_tc_topk(
        embeddings, adapted_embeddings, m_list.astype(jnp.int32), n, d, blk)

    nw = 32
    rows_per_w = n // nw
    ppw = rows_per_w * TOPK
    idxq = idxs_t.reshape(TOPK * n)
    valq = vals_t.reshape(TOPK * n)

    mesh = plsc.VectorSubcoreMesh(core_axis_name="c", subcore_axis_name="s")
    sc = pl.kernel(
        functools.partial(_pairs_sc_kernel, d=d, n=n, topk=TOPK,
                          rows_per_w=rows_per_w),
        mesh=mesh,
        compiler_params=pltpu.CompilerParams(needs_layout_passes=False,
                                             use_tc_tiling_on_sc=False),
        out_type=(
            jax.ShapeDtypeStruct((nw, 16), jnp.float32),
            jax.ShapeDtypeStruct((nw, 16), jnp.float32),
        ),
        scratch_types=[
            pltpu.VMEM((rows_per_w, d), jnp.float32),
            pltpu.VMEM((ppw, d), jnp.float32),
            pltpu.VMEM((ppw,), jnp.int32),
            pltpu.VMEM((ppw,), jnp.float32),
            pltpu.VMEM((16,), jnp.float32),
            pltpu.VMEM((16,), jnp.float32),
            pltpu.SemaphoreType.DMA,
        ],
    )
    s_part, c_part = sc(a_masked, idxq, valq)

    loss = jnp.sum(s_part) / jnp.float32(n * n)
    return loss / jnp.sum(c_part)
